# Initial kernel scaffold; baseline (speedup 1.0000x reference)
import jax
import jax.numpy as jnp
from jax.experimental import pallas as pl


def kernel(query, query_pos, value, reference_points, score_tgt, foreground_pre_layer, spatial_shapes, level_start_index, in_proj_w, in_proj_b, out_proj_w, out_proj_b, pre_norm_w, pre_norm_b, vp_w, vp_b, so_w, so_b, aw_w, aw_b, op_w, op_b, norm1_w, norm1_b, l1_w, l1_b, l2_w, l2_b, norm2_w, norm2_b):
    raise NotImplementedError("write your pallas kernel here")


# speedup vs baseline: 65.2100x; 65.2100x over previous
import functools

import jax
import jax.numpy as jnp
import numpy as np
from jax import lax
from jax.experimental import pallas as pl
from jax.experimental.pallas import tpu as pltpu
from jax.experimental.pallas import tpu_sc as plsc

_SPATIAL = [(128, 128), (64, 64), (32, 32), (16, 16)]
_NLVL = 4
_NHEAD = 8
_NPTS = 4
_C = 256
_DFFN = 1024
_TOPK = 300
_NTOK = sum(h * w for h, w in _SPATIAL)
_DH = _C // _NHEAD
_NPAIR = _NTOK * _NHEAD
_PADQ = 384
_TILE = 256
_NTILE = _NTOK // _TILE

_LMAP = np.repeat(np.tile(np.arange(_NLVL), _NHEAD), _NPTS)
_HMAP = np.repeat(np.arange(_NHEAD), _NLVL * _NPTS)
_WCOL = np.array([_SPATIAL[l][1] for l in _LMAP], np.float32)
_HCOL = np.array([_SPATIAL[l][0] for l in _LMAP], np.float32)
_LSTART = np.cumsum([0] + [h * w for h, w in _SPATIAL])[:-1]
_LSCOL = np.array([_LSTART[l] for l in _LMAP], np.int32)
_GROUP = np.kron(np.eye(_NHEAD, dtype=np.float32),
                 np.ones((_NLVL * _NPTS, _NLVL * _NPTS), np.float32))


def _layer_norm(x, w, b):
    m = x.mean(-1, keepdims=True)
    v = ((x - m) ** 2).mean(-1, keepdims=True)
    return (x - m) * lax.rsqrt(v + 1e-5) * w + b


def _mha_body(st_ref, sp_ref, wqT_ref, wkT_ref, wvT_ref, bq_ref, bk_ref,
              bv_ref, owT_ref, ob_ref, pnw_ref, pnb_ref, o_ref):
    st = st_ref[...]
    qk = st + sp_ref[...]
    q_all = jnp.dot(qk, wqT_ref[...], preferred_element_type=jnp.float32) + bq_ref[...]
    k_all = jnp.dot(qk, wkT_ref[...], preferred_element_type=jnp.float32) + bk_ref[...]
    v_all = jnp.dot(st, wvT_ref[...], preferred_element_type=jnp.float32) + bv_ref[...]
    col = lax.broadcasted_iota(jnp.int32, (_PADQ, _PADQ), 1)
    scale = 1.0 / np.sqrt(_DH)
    outs = []
    for h in range(_NHEAD):
        sl = slice(h * _DH, (h + 1) * _DH)
        qh, kh, vh = q_all[:, sl], k_all[:, sl], v_all[:, sl]
        s = lax.dot_general(qh, kh, (((1,), (1,)), ((), ())),
                            preferred_element_type=jnp.float32) * scale
        s = jnp.where(col < _TOPK, s, -1e30)
        m = s.max(-1, keepdims=True)
        e = jnp.exp(s - m)
        p = e / e.sum(-1, keepdims=True)
        outs.append(jnp.dot(p, vh, preferred_element_type=jnp.float32))
    o = jnp.concatenate(outs, axis=1)
    tgt2 = jnp.dot(o, owT_ref[...], preferred_element_type=jnp.float32) + ob_ref[...]
    o_ref[...] = _layer_norm(st + tgt2, pnw_ref[...], pnb_ref[...])


def _proj_body(qu_ref, qp_ref, val_ref, rpx_ref, rpy_ref, vpT_ref, vpb_ref,
               soxT_ref, soxb_ref, soyT_ref, soyb_ref, awT_ref, awb_ref,
               grp_ref, wc_ref, hc_ref, lsc_ref, hmap_ref,
               v_ref, i0_ref, i1_ref, i2_ref, i3_ref,
               w0_ref, w1_ref, w2_ref, w3_ref):
    q = qu_ref[...] + qp_ref[...]
    v_ref[...] = jnp.dot(val_ref[...], vpT_ref[...],
                         preferred_element_type=jnp.float32) + vpb_ref[...]
    offx = jnp.dot(q, soxT_ref[...], preferred_element_type=jnp.float32) + soxb_ref[...]
    offy = jnp.dot(q, soyT_ref[...], preferred_element_type=jnp.float32) + soyb_ref[...]
    awl = jnp.dot(q, awT_ref[...], preferred_element_type=jnp.float32) + awb_ref[...]
    awl = awl - awl.max(-1, keepdims=True)
    e = jnp.exp(awl)
    ssum = jnp.dot(e, grp_ref[...], preferred_element_type=jnp.float32)
    aw = e / ssum

    wc = wc_ref[...]
    hc = hc_ref[...]
    wci = wc.astype(jnp.int32)
    hci = hc.astype(jnp.int32)
    lsc = lsc_ref[...]
    hmap = hmap_ref[...]

    x = rpx_ref[...] * wc + offx - 0.5
    y = rpy_ref[...] * hc + offy - 0.5
    x0f = jnp.floor(x)
    y0f = jnp.floor(y)
    lx = x - x0f
    ly = y - y0f
    x0 = x0f.astype(jnp.int32)
    y0 = y0f.astype(jnp.int32)

    taps = [(0, 0, (1 - lx) * (1 - ly)), (1, 0, lx * (1 - ly)),
            (0, 1, (1 - lx) * ly), (1, 1, lx * ly)]
    irefs = [i0_ref, i1_ref, i2_ref, i3_ref]
    wrefs = [w0_ref, w1_ref, w2_ref, w3_ref]
    for (dx, dy, wbil), iref, wref in zip(taps, irefs, wrefs):
        xi = x0 + dx
        yi = y0 + dy
        valid = ((xi >= 0) & (xi < wci) & (yi >= 0) & (yi < hci))
        xcl = jnp.clip(xi, 0, wci - 1)
        ycl = jnp.clip(yi, 0, hci - 1)
        row = (lsc + ycl * wci + xcl) * _NHEAD + hmap
        iref[...] = row
        wref[...] = wbil * aw * valid.astype(jnp.float32)


def _post_body(s_ref, qu_ref, opT_ref, opb_ref, n1w_ref, n1b_ref,
               l1T_ref, l1b_ref, l2T_ref, l2b_ref, n2w_ref, n2b_ref, o_ref):
    src2 = jnp.dot(s_ref[...], opT_ref[...],
                   preferred_element_type=jnp.float32) + opb_ref[...]
    q1 = _layer_norm(qu_ref[...] + src2, n1w_ref[...], n1b_ref[...])
    hdn = jnp.maximum(
        jnp.dot(q1, l1T_ref[...], preferred_element_type=jnp.float32) + l1b_ref[...],
        0.0)
    y = jnp.dot(hdn, l2T_ref[...], preferred_element_type=jnp.float32) + l2b_ref[...]
    o_ref[...] = _layer_norm(q1 + y, n2w_ref[...], n2b_ref[...])


_CP = 32


def _sc_sample(v2, idxs, wgts):
    info = plsc.get_sparse_core_info()
    nw = info.num_cores * info.num_subcores
    pairs_per_w = _NPAIR // nw
    nchunk = pairs_per_w // _CP
    mesh = plsc.VectorSubcoreMesh(core_axis_name="c", subcore_axis_name="s")

    @functools.partial(
        pl.kernel, mesh=mesh,
        compiler_params=pltpu.CompilerParams(use_tc_tiling_on_sc=False),
        out_type=jax.ShapeDtypeStruct((_NPAIR, _DH), jnp.float32),
        scratch_types=(
            [pltpu.VMEM((_CP * 16,), jnp.int32) for _ in range(4)]
            + [pltpu.VMEM((_CP * 16,), jnp.float32) for _ in range(4)]
            + [pltpu.VMEM((_CP * 16, _DH), jnp.float32) for _ in range(4)]
            + [pltpu.VMEM((_CP, _DH), jnp.float32), pltpu.SemaphoreType.DMA]),
    )
    def k(v_hbm, i0, i1, i2, i3, w0, w1, w2, w3, out_hbm,
          iv0, iv1, iv2, iv3, wv0, wv1, wv2, wv3,
          rv0, rv1, rv2, rv3, ov, sem):
        wid = lax.axis_index("s") * info.num_cores + lax.axis_index("c")
        wbase = wid * pairs_per_w
        ihbm = [i0, i1, i2, i3]
        whbm = [w0, w1, w2, w3]
        ivs = [iv0, iv1, iv2, iv3]
        wvs = [wv0, wv1, wv2, wv3]
        rvs = [rv0, rv1, rv2, rv3]

        def chunk(c, carry):
            p0 = wbase + c * _CP
            t0 = p0 * 16
            for kk in range(4):
                pltpu.sync_copy(ihbm[kk].at[pl.ds(t0, _CP * 16)], ivs[kk])
                pltpu.sync_copy(whbm[kk].at[pl.ds(t0, _CP * 16)], wvs[kk])
            for kk in range(4):
                pltpu.async_copy(v_hbm.at[ivs[kk]], rvs[kk], sem).wait()

            dnums = lax.GatherDimensionNumbers(
                offset_dims=(), collapsed_slice_dims=(0,),
                start_index_map=(0,))

            def pair(p, carry2):
                acc0 = jnp.zeros((16,), jnp.float32)
                acc1 = jnp.zeros((16,), jnp.float32)
                for kk in range(4):
                    wvec = wvs[kk][pl.ds(p * 16, 16)]
                    for j in range(16):
                        ws = lax.gather(
                            wvec, jnp.full((16, 1), j, jnp.int32), dnums,
                            (1,),
                            mode=lax.GatherScatterMode.PROMISE_IN_BOUNDS)
                        r0 = rvs[kk][p * 16 + j, pl.ds(0, 16)]
                        r1 = rvs[kk][p * 16 + j, pl.ds(16, 16)]
                        acc0 = acc0 + ws * r0
                        acc1 = acc1 + ws * r1
                ov[p, pl.ds(0, 16)] = acc0
                ov[p, pl.ds(16, 16)] = acc1
                return carry2

            lax.fori_loop(0, _CP, pair, 0)
            pltpu.sync_copy(ov, out_hbm.at[pl.ds(p0, _CP)])
            return carry

        lax.fori_loop(0, nchunk, chunk, 0)

    return k(v2, *idxs, *wgts)


def _row(x):
    return x.reshape(1, -1)


def kernel(query, query_pos, value, reference_points, score_tgt,
           foreground_pre_layer, spatial_shapes, level_start_index,
           in_proj_w, in_proj_b, out_proj_w, out_proj_b,
           pre_norm_w, pre_norm_b, vp_w, vp_b, so_w, so_b,
           aw_w, aw_b, op_w, op_b, norm1_w, norm1_b,
           l1_w, l1_b, l2_w, l2_b, norm2_w, norm2_b):
    f32 = jnp.float32
    mc = score_tgt.max(-1) * foreground_pre_layer
    _, idx = lax.top_k(mc, _TOPK)
    idx3 = jnp.broadcast_to(idx[..., None], (1, _TOPK, _C))
    sel_t = jnp.take_along_axis(query, idx3, axis=1)[0]
    sel_p = jnp.take_along_axis(query_pos, idx3, axis=1)[0]
    pad = ((0, _PADQ - _TOPK), (0, 0))
    sel_t_p = jnp.pad(sel_t, pad)
    sel_p_p = jnp.pad(sel_p, pad)

    wq, wk, wv = jnp.split(in_proj_w, 3, axis=0)
    bq, bk, bv = jnp.split(in_proj_b, 3)
    spec = pl.BlockSpec((_PADQ, _C), lambda: (0, 0))
    wspec = pl.BlockSpec((_C, _C), lambda: (0, 0))
    bspec = pl.BlockSpec((1, _C), lambda: (0, 0))
    sel_out = pl.pallas_call(
        _mha_body,
        out_shape=jax.ShapeDtypeStruct((_PADQ, _C), f32),
        in_specs=[spec, spec, wspec, wspec, wspec, bspec, bspec, bspec,
                  wspec, bspec, bspec, bspec],
        out_specs=spec,
    )(sel_t_p, sel_p_p, wq.T, wk.T, wv.T, _row(bq), _row(bk), _row(bv),
      out_proj_w.T, _row(out_proj_b), _row(pre_norm_w), _row(pre_norm_b))

    query_upd = query.at[0, idx[0]].set(sel_out[:_TOPK])[0]

    rp = reference_points[0]
    rpx = rp[:, _LMAP, 0]
    rpy = rp[:, _LMAP, 1]
    base = (np.arange(_NHEAD * _NLVL * _NPTS) * 2)
    sox_w = so_w[base]
    soy_w = so_w[base + 1]
    sox_b = so_b[base]
    soy_b = so_b[base + 1]

    tspec = pl.BlockSpec((_TILE, _C), lambda i: (i, 0))
    hspec = pl.BlockSpec((_TILE, 128), lambda i: (i, 0))
    wspec2 = pl.BlockSpec((_C, _C), lambda i: (0, 0))
    w128 = pl.BlockSpec((_C, 128), lambda i: (0, 0))
    b256 = pl.BlockSpec((1, _C), lambda i: (0, 0))
    b128 = pl.BlockSpec((1, 128), lambda i: (0, 0))
    i128 = jax.ShapeDtypeStruct((_NTOK, 128), jnp.int32)
    f128 = jax.ShapeDtypeStruct((_NTOK, 128), f32)
    v_proj, i0, i1, i2, i3, w0, w1, w2, w3 = pl.pallas_call(
        _proj_body,
        grid=(_NTILE,),
        out_shape=[jax.ShapeDtypeStruct((_NTOK, _C), f32),
                   i128, i128, i128, i128, f128, f128, f128, f128],
        in_specs=[tspec, tspec, tspec, hspec, hspec, wspec2, b256,
                  w128, b128, w128, b128, w128, b128,
                  pl.BlockSpec((128, 128), lambda i: (0, 0)),
                  b128, b128, b128, b128],
        out_specs=[tspec, hspec, hspec, hspec, hspec,
                   hspec, hspec, hspec, hspec],
    )(query_upd, query_pos[0], value[0], rpx, rpy, vp_w.T, _row(vp_b),
      sox_w.T, _row(sox_b), soy_w.T, _row(soy_b), aw_w.T, _row(aw_b),
      jnp.asarray(_GROUP), jnp.asarray(_WCOL)[None, :],
      jnp.asarray(_HCOL)[None, :], jnp.asarray(_LSCOL)[None, :],
      jnp.asarray(_HMAP.astype(np.int32))[None, :])

    sampled = _sc_sample(
        v_proj.reshape(_NPAIR, _DH),
        [i0.reshape(-1), i1.reshape(-1), i2.reshape(-1), i3.reshape(-1)],
        [w0.reshape(-1), w1.reshape(-1), w2.reshape(-1), w3.reshape(-1)])

    wffn1 = pl.BlockSpec((_C, _DFFN), lambda i: (0, 0))
    wffn2 = pl.BlockSpec((_DFFN, _C), lambda i: (0, 0))
    bffn = pl.BlockSpec((1, _DFFN), lambda i: (0, 0))
    out = pl.pallas_call(
        _post_body,
        grid=(_NTILE,),
        out_shape=jax.ShapeDtypeStruct((_NTOK, _C), f32),
        in_specs=[tspec, tspec, wspec2, b256, b256, b256,
                  wffn1, bffn, wffn2, b256, b256, b256],
        out_specs=tspec,
    )(sampled.reshape(_NTOK, _C), query_upd, op_w.T, _row(op_b),
      _row(norm1_w), _row(norm1_b), l1_w.T, _row(l1_b), l2_w.T, _row(l2_b),
      _row(norm2_w), _row(norm2_b))
    return out[None]


# speedup vs baseline: 65.9395x; 1.0112x over previous
import functools

import jax
import jax.numpy as jnp
import numpy as np
from jax import lax
from jax.experimental import pallas as pl
from jax.experimental.pallas import tpu as pltpu
from jax.experimental.pallas import tpu_sc as plsc

_SPATIAL = [(128, 128), (64, 64), (32, 32), (16, 16)]
_NLVL = 4
_NHEAD = 8
_NPTS = 4
_C = 256
_DFFN = 1024
_TOPK = 300
_NTOK = sum(h * w for h, w in _SPATIAL)
_DH = _C // _NHEAD
_NPAIR = _NTOK * _NHEAD
_PADQ = 384
_TILE = 256
_NTILE = _NTOK // _TILE

_LMAP = np.repeat(np.tile(np.arange(_NLVL), _NHEAD), _NPTS)
_HMAP = np.repeat(np.arange(_NHEAD), _NLVL * _NPTS)
_WCOL = np.array([_SPATIAL[l][1] for l in _LMAP], np.float32)
_HCOL = np.array([_SPATIAL[l][0] for l in _LMAP], np.float32)
_LSTART = np.cumsum([0] + [h * w for h, w in _SPATIAL])[:-1]
_LSCOL = np.array([_LSTART[l] for l in _LMAP], np.int32)
_GROUP = np.kron(np.eye(_NHEAD, dtype=np.float32),
                 np.ones((_NLVL * _NPTS, _NLVL * _NPTS), np.float32))


def _layer_norm(x, w, b):
    m = x.mean(-1, keepdims=True)
    v = ((x - m) ** 2).mean(-1, keepdims=True)
    return (x - m) * lax.rsqrt(v + 1e-5) * w + b


def _mha_body(st_ref, sp_ref, wqT_ref, wkT_ref, wvT_ref, bq_ref, bk_ref,
              bv_ref, owT_ref, ob_ref, pnw_ref, pnb_ref, o_ref):
    st = st_ref[...]
    qk = st + sp_ref[...]
    q_all = jnp.dot(qk, wqT_ref[...], preferred_element_type=jnp.float32) + bq_ref[...]
    k_all = jnp.dot(qk, wkT_ref[...], preferred_element_type=jnp.float32) + bk_ref[...]
    v_all = jnp.dot(st, wvT_ref[...], preferred_element_type=jnp.float32) + bv_ref[...]
    col = lax.broadcasted_iota(jnp.int32, (_PADQ, _PADQ), 1)
    scale = 1.0 / np.sqrt(_DH)
    outs = []
    for h in range(_NHEAD):
        sl = slice(h * _DH, (h + 1) * _DH)
        qh, kh, vh = q_all[:, sl], k_all[:, sl], v_all[:, sl]
        s = lax.dot_general(qh, kh, (((1,), (1,)), ((), ())),
                            preferred_element_type=jnp.float32) * scale
        s = jnp.where(col < _TOPK, s, -1e30)
        m = s.max(-1, keepdims=True)
        e = jnp.exp(s - m)
        p = e / e.sum(-1, keepdims=True)
        outs.append(jnp.dot(p, vh, preferred_element_type=jnp.float32))
    o = jnp.concatenate(outs, axis=1)
    tgt2 = jnp.dot(o, owT_ref[...], preferred_element_type=jnp.float32) + ob_ref[...]
    o_ref[...] = _layer_norm(st + tgt2, pnw_ref[...], pnb_ref[...])


def _proj_body(qu_ref, qp_ref, val_ref, rpx_ref, rpy_ref, vpT_ref, vpb_ref,
               soxT_ref, soxb_ref, soyT_ref, soyb_ref, awT_ref, awb_ref,
               grp_ref, wc_ref, hc_ref, lsc_ref, hmap_ref,
               v_ref, i0_ref, i1_ref, i2_ref, i3_ref,
               w0_ref, w1_ref, w2_ref, w3_ref):
    q = qu_ref[...] + qp_ref[...]
    v_ref[...] = jnp.dot(val_ref[...], vpT_ref[...],
                         preferred_element_type=jnp.float32) + vpb_ref[...]
    offx = jnp.dot(q, soxT_ref[...], preferred_element_type=jnp.float32) + soxb_ref[...]
    offy = jnp.dot(q, soyT_ref[...], preferred_element_type=jnp.float32) + soyb_ref[...]
    awl = jnp.dot(q, awT_ref[...], preferred_element_type=jnp.float32) + awb_ref[...]
    awl = awl - awl.max(-1, keepdims=True)
    e = jnp.exp(awl)
    ssum = jnp.dot(e, grp_ref[...], preferred_element_type=jnp.float32)
    aw = e / ssum

    wc = wc_ref[...]
    hc = hc_ref[...]
    wci = wc.astype(jnp.int32)
    hci = hc.astype(jnp.int32)
    lsc = lsc_ref[...]
    hmap = hmap_ref[...]

    x = rpx_ref[...] * wc + offx - 0.5
    y = rpy_ref[...] * hc + offy - 0.5
    x0f = jnp.floor(x)
    y0f = jnp.floor(y)
    lx = x - x0f
    ly = y - y0f
    x0 = x0f.astype(jnp.int32)
    y0 = y0f.astype(jnp.int32)

    taps = [(0, 0, (1 - lx) * (1 - ly)), (1, 0, lx * (1 - ly)),
            (0, 1, (1 - lx) * ly), (1, 1, lx * ly)]
    irefs = [i0_ref, i1_ref, i2_ref, i3_ref]
    wrefs = [w0_ref, w1_ref, w2_ref, w3_ref]
    for (dx, dy, wbil), iref, wref in zip(taps, irefs, wrefs):
        xi = x0 + dx
        yi = y0 + dy
        valid = ((xi >= 0) & (xi < wci) & (yi >= 0) & (yi < hci))
        xcl = jnp.clip(xi, 0, wci - 1)
        ycl = jnp.clip(yi, 0, hci - 1)
        row = (lsc + ycl * wci + xcl) * _NHEAD + hmap
        iref[...] = row
        wref[...] = wbil * aw * valid.astype(jnp.float32)


def _post_body(s_ref, qu_ref, opT_ref, opb_ref, n1w_ref, n1b_ref,
               l1T_ref, l1b_ref, l2T_ref, l2b_ref, n2w_ref, n2b_ref, o_ref):
    src2 = jnp.dot(s_ref[...], opT_ref[...],
                   preferred_element_type=jnp.float32) + opb_ref[...]
    q1 = _layer_norm(qu_ref[...] + src2, n1w_ref[...], n1b_ref[...])
    hdn = jnp.maximum(
        jnp.dot(q1, l1T_ref[...], preferred_element_type=jnp.float32) + l1b_ref[...],
        0.0)
    y = jnp.dot(hdn, l2T_ref[...], preferred_element_type=jnp.float32) + l2b_ref[...]
    o_ref[...] = _layer_norm(q1 + y, n2w_ref[...], n2b_ref[...])


_CP = 16


def _sc_sample(v2, idxs, wgts):
    info = plsc.get_sparse_core_info()
    nw = info.num_cores * info.num_subcores
    pairs_per_w = _NPAIR // nw
    nchunk = pairs_per_w // _CP
    mesh = plsc.VectorSubcoreMesh(core_axis_name="c", subcore_axis_name="s")

    vbuf = (
        [pltpu.VMEM((_CP * 16,), jnp.int32) for _ in range(4)]
        + [pltpu.VMEM((_CP * 16,), jnp.float32) for _ in range(4)]
        + [pltpu.VMEM((_CP * 16, _DH), jnp.float32) for _ in range(4)]
        + [pltpu.SemaphoreType.DMA])

    @functools.partial(
        pl.kernel, mesh=mesh,
        compiler_params=pltpu.CompilerParams(use_tc_tiling_on_sc=False),
        out_type=jax.ShapeDtypeStruct((_NPAIR, _DH), jnp.float32),
        scratch_types=(vbuf + vbuf + [pltpu.VMEM((_CP, _DH), jnp.float32)]),
    )
    def k(v_hbm, i0, i1, i2, i3, w0, w1, w2, w3, out_hbm, *scr):
        wid = lax.axis_index("s") * info.num_cores + lax.axis_index("c")
        wbase = wid * pairs_per_w
        ihbm = [i0, i1, i2, i3]
        whbm = [w0, w1, w2, w3]
        bufs = []
        for b in range(2):
            sc = scr[b * 13:(b + 1) * 13]
            bufs.append((sc[0:4], sc[4:8], sc[8:12], sc[12]))
        ov = scr[26]

        def issue(c, buf):
            ivs, wvs, rvs, sem = buf
            t0 = (wbase + c * _CP) * 16
            for kk in range(4):
                pltpu.sync_copy(ihbm[kk].at[pl.ds(t0, _CP * 16)], ivs[kk])
                pltpu.sync_copy(whbm[kk].at[pl.ds(t0, _CP * 16)], wvs[kk])
            for kk in range(4):
                pltpu.async_copy(v_hbm.at[ivs[kk]], rvs[kk], sem)

        def drain(buf):
            ivs, _, rvs, sem = buf
            for kk in range(4):
                pltpu.make_async_copy(v_hbm.at[ivs[kk]], rvs[kk], sem).wait()

        dnums = lax.GatherDimensionNumbers(
            offset_dims=(), collapsed_slice_dims=(0,), start_index_map=(0,))

        def compute(c, buf):
            _, wvs, rvs, _ = buf

            def pair(p, carry2):
                acc0 = jnp.zeros((16,), jnp.float32)
                acc1 = jnp.zeros((16,), jnp.float32)
                for kk in range(4):
                    wvec = wvs[kk][pl.ds(p * 16, 16)]
                    for j in range(16):
                        ws = lax.gather(
                            wvec, jnp.full((16, 1), j, jnp.int32), dnums,
                            (1,),
                            mode=lax.GatherScatterMode.PROMISE_IN_BOUNDS)
                        r0 = rvs[kk][p * 16 + j, pl.ds(0, 16)]
                        r1 = rvs[kk][p * 16 + j, pl.ds(16, 16)]
                        acc0 = acc0 + ws * r0
                        acc1 = acc1 + ws * r1
                ov[p, pl.ds(0, 16)] = acc0
                ov[p, pl.ds(16, 16)] = acc1
                return carry2

            lax.fori_loop(0, _CP, pair, 0)
            pltpu.sync_copy(ov, out_hbm.at[pl.ds(wbase + c * _CP, _CP)])

        issue(0, bufs[0])

        def outer(i, carry):
            c0 = i * 2
            for b in range(2):
                c = c0 + b
                drain(bufs[b])

                @pl.when(c + 1 < nchunk)
                def _():
                    issue(c + 1, bufs[1 - b])

                compute(c, bufs[b])
            return carry

        lax.fori_loop(0, nchunk // 2, outer, 0)

    return k(v2, *idxs, *wgts)


def _row(x):
    return x.reshape(1, -1)


def kernel(query, query_pos, value, reference_points, score_tgt,
           foreground_pre_layer, spatial_shapes, level_start_index,
           in_proj_w, in_proj_b, out_proj_w, out_proj_b,
           pre_norm_w, pre_norm_b, vp_w, vp_b, so_w, so_b,
           aw_w, aw_b, op_w, op_b, norm1_w, norm1_b,
           l1_w, l1_b, l2_w, l2_b, norm2_w, norm2_b):
    f32 = jnp.float32
    mc = score_tgt.max(-1) * foreground_pre_layer
    _, idx = lax.top_k(mc, _TOPK)
    idx3 = jnp.broadcast_to(idx[..., None], (1, _TOPK, _C))
    sel_t = jnp.take_along_axis(query, idx3, axis=1)[0]
    sel_p = jnp.take_along_axis(query_pos, idx3, axis=1)[0]
    pad = ((0, _PADQ - _TOPK), (0, 0))
    sel_t_p = jnp.pad(sel_t, pad)
    sel_p_p = jnp.pad(sel_p, pad)

    wq, wk, wv = jnp.split(in_proj_w, 3, axis=0)
    bq, bk, bv = jnp.split(in_proj_b, 3)
    spec = pl.BlockSpec((_PADQ, _C), lambda: (0, 0))
    wspec = pl.BlockSpec((_C, _C), lambda: (0, 0))
    bspec = pl.BlockSpec((1, _C), lambda: (0, 0))
    sel_out = pl.pallas_call(
        _mha_body,
        out_shape=jax.ShapeDtypeStruct((_PADQ, _C), f32),
        in_specs=[spec, spec, wspec, wspec, wspec, bspec, bspec, bspec,
                  wspec, bspec, bspec, bspec],
        out_specs=spec,
    )(sel_t_p, sel_p_p, wq.T, wk.T, wv.T, _row(bq), _row(bk), _row(bv),
      out_proj_w.T, _row(out_proj_b), _row(pre_norm_w), _row(pre_norm_b))

    query_upd = query.at[0, idx[0]].set(sel_out[:_TOPK])[0]

    rp = reference_points[0]
    rpx = rp[:, _LMAP, 0]
    rpy = rp[:, _LMAP, 1]
    base = (np.arange(_NHEAD * _NLVL * _NPTS) * 2)
    sox_w = so_w[base]
    soy_w = so_w[base + 1]
    sox_b = so_b[base]
    soy_b = so_b[base + 1]

    tspec = pl.BlockSpec((_TILE, _C), lambda i: (i, 0))
    hspec = pl.BlockSpec((_TILE, 128), lambda i: (i, 0))
    wspec2 = pl.BlockSpec((_C, _C), lambda i: (0, 0))
    w128 = pl.BlockSpec((_C, 128), lambda i: (0, 0))
    b256 = pl.BlockSpec((1, _C), lambda i: (0, 0))
    b128 = pl.BlockSpec((1, 128), lambda i: (0, 0))
    i128 = jax.ShapeDtypeStruct((_NTOK, 128), jnp.int32)
    f128 = jax.ShapeDtypeStruct((_NTOK, 128), f32)
    v_proj, i0, i1, i2, i3, w0, w1, w2, w3 = pl.pallas_call(
        _proj_body,
        grid=(_NTILE,),
        out_shape=[jax.ShapeDtypeStruct((_NTOK, _C), f32),
                   i128, i128, i128, i128, f128, f128, f128, f128],
        in_specs=[tspec, tspec, tspec, hspec, hspec, wspec2, b256,
                  w128, b128, w128, b128, w128, b128,
                  pl.BlockSpec((128, 128), lambda i: (0, 0)),
                  b128, b128, b128, b128],
        out_specs=[tspec, hspec, hspec, hspec, hspec,
                   hspec, hspec, hspec, hspec],
    )(query_upd, query_pos[0], value[0], rpx, rpy, vp_w.T, _row(vp_b),
      sox_w.T, _row(sox_b), soy_w.T, _row(soy_b), aw_w.T, _row(aw_b),
      jnp.asarray(_GROUP), jnp.asarray(_WCOL)[None, :],
      jnp.asarray(_HCOL)[None, :], jnp.asarray(_LSCOL)[None, :],
      jnp.asarray(_HMAP.astype(np.int32))[None, :])

    sampled = _sc_sample(
        v_proj.reshape(_NPAIR, _DH),
        [i0.reshape(-1), i1.reshape(-1), i2.reshape(-1), i3.reshape(-1)],
        [w0.reshape(-1), w1.reshape(-1), w2.reshape(-1), w3.reshape(-1)])

    wffn1 = pl.BlockSpec((_C, _DFFN), lambda i: (0, 0))
    wffn2 = pl.BlockSpec((_DFFN, _C), lambda i: (0, 0))
    bffn = pl.BlockSpec((1, _DFFN), lambda i: (0, 0))
    out = pl.pallas_call(
        _post_body,
        grid=(_NTILE,),
        out_shape=jax.ShapeDtypeStruct((_NTOK, _C), f32),
        in_specs=[tspec, tspec, wspec2, b256, b256, b256,
                  wffn1, bffn, wffn2, b256, b256, b256],
        out_specs=tspec,
    )(sampled.reshape(_NTOK, _C), query_upd, op_w.T, _row(op_b),
      _row(norm1_w), _row(norm1_b), l1_w.T, _row(l1_b), l2_w.T, _row(l2_b),
      _row(norm2_w), _row(norm2_b))
    return out[None]
